# baseline (speedup 1.0000x reference)
import functools

import jax
import jax.numpy as jnp
import numpy as np
from jax import lax
from jax.experimental import pallas as pl
from jax.experimental.pallas import tpu as pltpu
from jax.experimental.pallas import tpu_sc as plsc

_NC = 2
_NS = 16
_NW = _NC * _NS

_QT = 1024
_CH = 32


def _nn_block(m_per, b0, q_ref, k_ref,
              i0_ref, i1_ref, i2_ref, d0_ref, d1_ref, d2_ref):
    b = pl.program_id(0) + b0
    d2 = None
    for d in range(3):
        kc = k_ref[:, d:d + 1]
        qr = q_ref[d:d + 1, :]
        diff = kc - qr
        sq = diff * diff
        d2 = sq if d2 is None else d2 + sq

    iota_f = lax.broadcasted_iota(jnp.int32, d2.shape, 0).astype(jnp.float32)
    inf = jnp.float32(np.inf)
    off = b * m_per
    idx_refs = (i0_ref, i1_ref, i2_ref)
    dist_refs = (d0_ref, d1_ref, d2_ref)
    for j in range(3):
        m = jnp.min(d2, axis=0, keepdims=True)
        cand = jnp.where(d2 == m, iota_f, inf)
        i_f = jnp.min(cand, axis=0, keepdims=True)
        dist_refs[j][0:1, :] = m
        idx_refs[j][0:1, :] = i_f.astype(jnp.int32) + off
        if j < 2:
            d2 = jnp.where(cand == i_f, inf, d2)


def _three_nn_tc(num_batches, m_per, n_per, b0):
    n_total = num_batches * n_per
    n_tiles = n_per // _QT
    grid = (num_batches, n_tiles)
    plane = pl.BlockSpec((1, _QT), lambda b, t: (0, b * n_tiles + t))
    return pl.pallas_call(
        functools.partial(_nn_block, m_per, b0),
        grid=grid,
        in_specs=[
            pl.BlockSpec((3, _QT), lambda b, t: (0, b * n_tiles + t)),
            pl.BlockSpec((m_per, 3), lambda b, t: (b + b0, 0)),
        ],
        out_specs=[plane] * 6,
        out_shape=[jax.ShapeDtypeStruct((1, n_total), jnp.int32)] * 3
        + [jax.ShapeDtypeStruct((1, n_total), jnp.float32)] * 3,
    )


def _interp_sc(n_total, c):
    qw = n_total // _NW
    n_chunks = qw // _CH
    n_pairs = n_chunks // 2
    mesh = plsc.VectorSubcoreMesh(core_axis_name="c", subcore_axis_name="s")

    @functools.partial(
        pl.kernel,
        out_type=jax.ShapeDtypeStruct((n_total, c), jnp.float32),
        mesh=mesh,
        scratch_types=[
            pltpu.VMEM((qw,), jnp.int32),
            pltpu.VMEM((qw,), jnp.int32),
            pltpu.VMEM((qw,), jnp.int32),
            pltpu.VMEM((qw,), jnp.float32),
            pltpu.VMEM((qw,), jnp.float32),
            pltpu.VMEM((qw,), jnp.float32),
            pltpu.VMEM((_CH, c), jnp.float32),
            pltpu.VMEM((_CH, c), jnp.float32),
            pltpu.VMEM((_CH, c), jnp.float32),
            pltpu.VMEM((_CH, c), jnp.float32),
            pltpu.VMEM((_CH, c), jnp.float32),
            pltpu.VMEM((_CH, c), jnp.float32),
            pltpu.VMEM((_CH, c), jnp.float32),
            pltpu.VMEM((_CH, c), jnp.float32),
            [pltpu.SemaphoreType.DMA] * 6,
            [pltpu.SemaphoreType.DMA] * 2,
        ],
    )
    def interp(i0_hbm, i1_hbm, i2_hbm, d0_hbm, d1_hbm, d2_hbm, feat_hbm,
               out_hbm, i0_v, i1_v, i2_v, w0_v, w1_v, w2_v,
               ra0, ra1, ra2, rb0, rb1, rb2, out_a, out_b, gsems, osems):
        wid = lax.axis_index("s") * _NC + lax.axis_index("c")
        qbase0 = pl.multiple_of(wid * qw, _CH)
        idx_refs = (i0_v, i1_v, i2_v)
        row_refs = ((ra0, ra1, ra2), (rb0, rb1, rb2))
        out_refs = (out_a, out_b)

        for src, dst in ((i0_hbm, i0_v), (i1_hbm, i1_v), (i2_hbm, i2_v),
                         (d0_hbm, w0_v), (d1_hbm, w1_v), (d2_hbm, w2_v)):
            pltpu.sync_copy(src.at[pl.ds(qbase0, qw)], dst)

        @plsc.parallel_loop(0, qw // 16, 1, unroll=2)
        def _(g):
            sl = pl.ds(pl.multiple_of(g * 16, 16), 16)
            r0 = 1.0 / (w0_v[sl] + 1e-8)
            r1 = 1.0 / (w1_v[sl] + 1e-8)
            r2 = 1.0 / (w2_v[sl] + 1e-8)
            s = r0 + r1 + r2
            w0_v[sl] = r0 / s
            w1_v[sl] = r1 / s
            w2_v[sl] = r2 / s

        def start_gathers(t, slot):
            base = pl.multiple_of(t * _CH, _CH)
            for j in range(3):
                pltpu.async_copy(
                    feat_hbm.at[idx_refs[j].at[pl.ds(base, _CH)]],
                    row_refs[slot][j], gsems[slot * 3 + j])

        def wait_gathers(t, slot):
            base = pl.multiple_of(t * _CH, _CH)
            for j in range(3):
                pltpu.make_async_copy(
                    feat_hbm.at[idx_refs[j].at[pl.ds(base, _CH)]],
                    row_refs[slot][j], gsems[slot * 3 + j]).wait()

        def out_slice(t):
            return out_hbm.at[pl.ds(pl.multiple_of(qbase0 + t * _CH, _CH),
                                    _CH)]

        def do_chunk(p, t, slot):
            wait_gathers(t, slot)
            r0r, r1r, r2r = row_refs[slot]
            outr = out_refs[slot]

            @pl.when(p > 0)
            def _():
                pltpu.make_async_copy(outr, out_slice(t), osems[slot]).wait()

            tch = t * _CH

            def q_body(i, _):
                qi = tch + i
                base = pl.multiple_of((qi // 16) * 16, 16)
                lane = jnp.full((16,), qi - base, jnp.int32)
                w0s = w0_v[pl.ds(base, 16)].at[lane].get(
                    mode="promise_in_bounds")
                w1s = w1_v[pl.ds(base, 16)].at[lane].get(
                    mode="promise_in_bounds")
                w2s = w2_v[pl.ds(base, 16)].at[lane].get(
                    mode="promise_in_bounds")
                for g in range(c // 16):
                    sl = pl.ds(g * 16, 16)
                    acc = r0r[i, sl] * w0s
                    acc = acc + r1r[i, sl] * w1s
                    acc = acc + r2r[i, sl] * w2s
                    outr[i, sl] = acc
                return 0

            lax.fori_loop(0, _CH, q_body, 0)
            pltpu.async_copy(outr, out_slice(t), osems[slot])

        def pair_body(p, _):
            t0 = 2 * p
            start_gathers(t0 + 1, 1)
            do_chunk(p, t0, 0)

            @pl.when(p + 1 < n_pairs)
            def _():
                start_gathers(t0 + 2, 0)

            do_chunk(p, t0 + 1, 1)
            return 0

        start_gathers(0, 0)
        lax.fori_loop(0, n_pairs, pair_body, 0)
        for slot, t in ((0, n_chunks - 2), (1, n_chunks - 1)):
            pltpu.make_async_copy(
                out_refs[slot], out_slice(t), osems[slot]).wait()

    return interp


def kernel(xyz, xyz_batch_cnt, new_xyz, new_xyz_batch_cnt, features):
    num_batches = xyz_batch_cnt.shape[0]
    m_per = xyz.shape[0] // num_batches
    n_per = new_xyz.shape[0] // num_batches
    n_total = new_xyz.shape[0]
    c = features.shape[1]

    nb1 = min(num_batches, max(1, (num_batches * 5) // 8))
    groups = [(0, nb1)]
    if nb1 < num_batches:
        groups.append((nb1, num_batches - nb1))
    outs = []
    for b0, nb in groups:
        q_t = new_xyz[b0 * n_per:(b0 + nb) * n_per].T
        planes = _three_nn_tc(nb, m_per, n_per, b0)(q_t, xyz)
        outs.append(_interp_sc(nb * n_per, c)(
            *[p.reshape(-1) for p in planes], features))
    return jnp.concatenate(outs, axis=0) if len(outs) > 1 else outs[0]


# speedup vs baseline: 1.0325x; 1.0325x over previous
import functools

import jax
import jax.numpy as jnp
import numpy as np
from jax import lax
from jax.experimental import pallas as pl
from jax.experimental.pallas import tpu as pltpu
from jax.experimental.pallas import tpu_sc as plsc

_NC = 2
_NS = 16
_NW = _NC * _NS

_QT = 1024
_CH = 32


def _nn_block(m_per, b0, q_ref, k_ref,
              i0_ref, i1_ref, i2_ref, d0_ref, d1_ref, d2_ref):
    b = pl.program_id(0) + b0
    d2 = None
    for d in range(3):
        kc = k_ref[:, d:d + 1]
        qr = q_ref[d:d + 1, :]
        diff = kc - qr
        sq = diff * diff
        d2 = sq if d2 is None else d2 + sq

    iota_f = lax.broadcasted_iota(jnp.int32, d2.shape, 0).astype(jnp.float32)
    inf = jnp.float32(np.inf)
    off = b * m_per
    idx_refs = (i0_ref, i1_ref, i2_ref)
    dist_refs = (d0_ref, d1_ref, d2_ref)
    for j in range(3):
        m = jnp.min(d2, axis=0, keepdims=True)
        cand = jnp.where(d2 == m, iota_f, inf)
        i_f = jnp.min(cand, axis=0, keepdims=True)
        dist_refs[j][0:1, :] = m
        idx_refs[j][0:1, :] = i_f.astype(jnp.int32) + off
        if j < 2:
            d2 = jnp.where(cand == i_f, inf, d2)


def _three_nn_tc(num_batches, m_per, n_per, b0):
    n_total = num_batches * n_per
    n_tiles = n_per // _QT
    grid = (num_batches, n_tiles)
    plane = pl.BlockSpec((1, _QT), lambda b, t: (0, b * n_tiles + t))
    return pl.pallas_call(
        functools.partial(_nn_block, m_per, b0),
        grid=grid,
        in_specs=[
            pl.BlockSpec((3, _QT), lambda b, t: (0, b * n_tiles + t)),
            pl.BlockSpec((m_per, 3), lambda b, t: (b + b0, 0)),
        ],
        out_specs=[plane] * 6,
        out_shape=[jax.ShapeDtypeStruct((1, n_total), jnp.int32)] * 3
        + [jax.ShapeDtypeStruct((1, n_total), jnp.float32)] * 3,
    )


def _interp_sc(n_total, c):
    qw = n_total // _NW
    n_chunks = qw // _CH
    n_pairs = n_chunks // 2
    mesh = plsc.VectorSubcoreMesh(core_axis_name="c", subcore_axis_name="s")

    @functools.partial(
        pl.kernel,
        out_type=jax.ShapeDtypeStruct((n_total, c), jnp.float32),
        mesh=mesh,
        scratch_types=[
            pltpu.VMEM((qw,), jnp.int32),
            pltpu.VMEM((qw,), jnp.int32),
            pltpu.VMEM((qw,), jnp.int32),
            pltpu.VMEM((qw,), jnp.float32),
            pltpu.VMEM((qw,), jnp.float32),
            pltpu.VMEM((qw,), jnp.float32),
            pltpu.VMEM((_CH, c), jnp.float32),
            pltpu.VMEM((_CH, c), jnp.float32),
            pltpu.VMEM((_CH, c), jnp.float32),
            pltpu.VMEM((_CH, c), jnp.float32),
            pltpu.VMEM((_CH, c), jnp.float32),
            pltpu.VMEM((_CH, c), jnp.float32),
            pltpu.VMEM((_CH, c), jnp.float32),
            pltpu.VMEM((_CH, c), jnp.float32),
            [pltpu.SemaphoreType.DMA] * 6,
            [pltpu.SemaphoreType.DMA] * 2,
        ],
    )
    def interp(i0_hbm, i1_hbm, i2_hbm, d0_hbm, d1_hbm, d2_hbm, feat_hbm,
               out_hbm, i0_v, i1_v, i2_v, w0_v, w1_v, w2_v,
               ra0, ra1, ra2, rb0, rb1, rb2, out_a, out_b, gsems, osems):
        wid = lax.axis_index("s") * _NC + lax.axis_index("c")
        qbase0 = pl.multiple_of(wid * qw, _CH)
        idx_refs = (i0_v, i1_v, i2_v)
        row_refs = ((ra0, ra1, ra2), (rb0, rb1, rb2))
        out_refs = (out_a, out_b)

        for src, dst in ((i0_hbm, i0_v), (i1_hbm, i1_v), (i2_hbm, i2_v),
                         (d0_hbm, w0_v), (d1_hbm, w1_v), (d2_hbm, w2_v)):
            pltpu.sync_copy(src.at[pl.ds(qbase0, qw)], dst)

        @plsc.parallel_loop(0, qw // 16, 1, unroll=2)
        def _(g):
            sl = pl.ds(pl.multiple_of(g * 16, 16), 16)
            r0 = 1.0 / (w0_v[sl] + 1e-8)
            r1 = 1.0 / (w1_v[sl] + 1e-8)
            r2 = 1.0 / (w2_v[sl] + 1e-8)
            s = r0 + r1 + r2
            w0_v[sl] = r0 / s
            w1_v[sl] = r1 / s
            w2_v[sl] = r2 / s

        def start_gathers(t, slot):
            base = pl.multiple_of(t * _CH, _CH)
            for j in range(3):
                pltpu.async_copy(
                    feat_hbm.at[idx_refs[j].at[pl.ds(base, _CH)]],
                    row_refs[slot][j], gsems[slot * 3 + j])

        def wait_gathers(t, slot):
            base = pl.multiple_of(t * _CH, _CH)
            for j in range(3):
                pltpu.make_async_copy(
                    feat_hbm.at[idx_refs[j].at[pl.ds(base, _CH)]],
                    row_refs[slot][j], gsems[slot * 3 + j]).wait()

        def out_slice(t):
            return out_hbm.at[pl.ds(pl.multiple_of(qbase0 + t * _CH, _CH),
                                    _CH)]

        def do_chunk(p, t, slot):
            wait_gathers(t, slot)
            r0r, r1r, r2r = row_refs[slot]
            outr = out_refs[slot]

            @pl.when(p > 0)
            def _():
                pltpu.make_async_copy(outr, out_slice(t), osems[slot]).wait()

            tch = t * _CH

            def q_body(i, _):
                qi = tch + i
                base = pl.multiple_of((qi // 16) * 16, 16)
                lane = jnp.full((16,), qi - base, jnp.int32)
                w0s = w0_v[pl.ds(base, 16)].at[lane].get(
                    mode="promise_in_bounds")
                w1s = w1_v[pl.ds(base, 16)].at[lane].get(
                    mode="promise_in_bounds")
                w2s = w2_v[pl.ds(base, 16)].at[lane].get(
                    mode="promise_in_bounds")
                for g in range(c // 16):
                    sl = pl.ds(g * 16, 16)
                    acc = r0r[i, sl] * w0s
                    acc = acc + r1r[i, sl] * w1s
                    acc = acc + r2r[i, sl] * w2s
                    outr[i, sl] = acc
                return 0

            lax.fori_loop(0, _CH, q_body, 0)
            pltpu.async_copy(outr, out_slice(t), osems[slot])

        def pair_body(p, _):
            t0 = 2 * p
            start_gathers(t0 + 1, 1)
            do_chunk(p, t0, 0)

            @pl.when(p + 1 < n_pairs)
            def _():
                start_gathers(t0 + 2, 0)

            do_chunk(p, t0 + 1, 1)
            return 0

        start_gathers(0, 0)
        lax.fori_loop(0, n_pairs, pair_body, 0)
        for slot, t in ((0, n_chunks - 2), (1, n_chunks - 1)):
            pltpu.make_async_copy(
                out_refs[slot], out_slice(t), osems[slot]).wait()

    return interp


def kernel(xyz, xyz_batch_cnt, new_xyz, new_xyz_batch_cnt, features):
    num_batches = xyz_batch_cnt.shape[0]
    m_per = xyz.shape[0] // num_batches
    n_per = new_xyz.shape[0] // num_batches
    n_total = new_xyz.shape[0]
    c = features.shape[1]

    nb1 = num_batches // 2
    groups = ((0, nb1), (nb1, num_batches - nb1)) if nb1 else \
        ((0, num_batches),)
    outs = []
    for b0, nb in groups:
        q_t = new_xyz[b0 * n_per:(b0 + nb) * n_per].T
        planes = _three_nn_tc(nb, m_per, n_per, b0)(q_t, xyz)
        outs.append(_interp_sc(nb * n_per, c)(
            *[p.reshape(-1) for p in planes], features))
    return jnp.concatenate(outs, axis=0) if len(outs) > 1 else outs[0]
